# baseline (speedup 1.0000x reference)
import jax
import jax.numpy as jnp
from jax import lax
from jax.experimental import pallas as pl
from jax.experimental.pallas import tpu as pltpu
from jax.experimental.pallas import tpu_sc as plsc

B = 128
V = 100000
N_FEAT = 25
TOPK = 10

N_OCT = 16
CW = 4992
N_CHUNKS = 10
HALF = CW * N_CHUNKS
REM_OFF = 2 * HALF
REM = V - REM_OFF
GROUP = 13
N_GROUPS = CW // (16 * GROUP)
CAND_CAP = 256

NEG_INF = float("-inf")


def _merge16(best_asc, v):
    v_desc, _ = plsc.sort_key_val(v, v, descending=True)
    h = jnp.maximum(best_asc, v_desc)
    h_asc, _ = plsc.sort_key_val(h, h, descending=False)
    return h_asc


def _tree_max(vals):
    while len(vals) > 1:
        nxt = [jnp.maximum(vals[i], vals[i + 1])
               for i in range(0, len(vals) - 1, 2)]
        if len(vals) % 2:
            nxt.append(vals[-1])
        vals = nxt
    return vals[0]


def _sc_body(item_hbm, feat_hbm, out1_hbm, out2_hbm,
             buf0, buf1, rbuf, cand, fbuf, bbuf, pbuf, o1buf, o2buf,
             shared, sem0, sem1):
    s_idx = lax.axis_index("s")
    c_idx = lax.axis_index("c")
    oct_i = c_idx * jnp.int32(8) + lax.div(s_idx, jnp.int32(2))
    h = lax.rem(s_idx, jnp.int32(2))
    lane = lax.iota(jnp.int32, 16)
    six = jnp.full((16,), 6, jnp.int32)
    col0 = h * jnp.int32(HALF)

    row0 = oct_i * jnp.int32(8)

    def chunk_slice(c):
        return item_hbm.at[pl.ds(row0, 8), pl.ds(col0 + c * jnp.int32(CW), CW)]

    def scan_group(vs, best, t_splat):
        acc = _tree_max(vs)
        cnt = plsc.all_reduce_population_count(acc > t_splat)
        pred = cnt[0] > 0

        def rescan(carry2):
            best2, t2 = carry2
            basev = jnp.zeros((16,), jnp.int32)
            for v in vs:
                msk = v > t2
                cum = plsc.cumsum(msk.astype(jnp.int32))
                n = plsc.all_reduce_population_count(msk)
                idx = jnp.maximum(basev + cum - 1, 0)
                plsc.store_scatter(cand, [idx], v, mask=msk)
                basev = basev + n
            ncand = basev[0]

            def wcond(carry3):
                i, _ = carry3
                return i * 16 < ncand

            def wbody(carry3):
                i, b = carry3
                w = cand[pl.ds(i * jnp.int32(16), 16)]
                valid = (i * 16 + lane) < ncand
                w = jnp.where(valid, w, NEG_INF)
                return i + 1, _merge16(b, w)

            _, best2 = lax.while_loop(wcond, wbody, (jnp.int32(0), best2))
            return best2, best2[six]

        return lax.cond(pred, rescan, lambda c2: c2, (best, t_splat))

    def process(buf, carry):
        def gbody(g, carry):
            bs, ts = list(carry[0]), list(carry[1])
            base = g * jnp.int32(16 * GROUP)
            for r in range(8):
                vs = [buf[r, pl.ds(base + jnp.int32(16 * j), 16)]
                      for j in range(GROUP)]
                bs[r], ts[r] = scan_group(vs, bs[r], ts[r])
            return tuple(bs), tuple(ts)

        return lax.fori_loop(jnp.int32(0), jnp.int32(N_GROUPS), gbody, carry)

    pltpu.make_async_copy(chunk_slice(jnp.int32(0)), buf0, sem0).start()
    pltpu.make_async_copy(chunk_slice(jnp.int32(1)), buf1, sem1).start()
    init = (tuple(jnp.full((16,), NEG_INF, jnp.float32) for _ in range(8)),
            tuple(jnp.full((16,), NEG_INF, jnp.float32) for _ in range(8)))

    def super_body(si, carry):
        c0 = si * jnp.int32(2)
        pltpu.make_async_copy(chunk_slice(c0), buf0, sem0).wait()
        carry = process(buf0, carry)
        nxt0 = jnp.minimum(c0 + 2, jnp.int32(8))
        pltpu.make_async_copy(chunk_slice(nxt0), buf0, sem0).start()
        pltpu.make_async_copy(chunk_slice(c0 + 1), buf1, sem1).wait()
        carry = process(buf1, carry)
        nxt1 = jnp.minimum(c0 + 3, jnp.int32(9))
        pltpu.make_async_copy(chunk_slice(nxt1), buf1, sem1).start()
        return carry

    carry = lax.fori_loop(jnp.int32(0), jnp.int32(5), super_body, init)
    pltpu.make_async_copy(chunk_slice(jnp.int32(8)), buf0, sem0).wait()
    pltpu.make_async_copy(chunk_slice(jnp.int32(9)), buf1, sem1).wait()

    pltpu.sync_copy(item_hbm.at[pl.ds(row0, 8), pl.ds(jnp.int32(REM_OFF), REM)],
                    rbuf)
    bs, ts = list(carry[0]), list(carry[1])
    rbase = h * jnp.int32(REM // 2)
    for r in range(8):
        vs = [rbuf[r, pl.ds(rbase + jnp.int32(16 * j), 16)]
              for j in range(REM // 32)]
        bs[r], ts[r] = scan_group(vs, bs[r], ts[r])

    for r in range(8):
        bbuf[r, :] = bs[r]
    pltpu.sync_copy(bbuf, shared.at[s_idx])
    plsc.subcore_barrier()

    @pl.when(h == 0)
    def _finalize():
        pltpu.sync_copy(shared.at[s_idx + 1], pbuf)
        pltpu.sync_copy(feat_hbm.at[pl.ds(row0, 8)], fbuf)
        for r in range(8):
            merged = _merge16(bs[r], pbuf[r, :])
            bdesc, _ = plsc.sort_key_val(merged, merged, descending=True)
            top = jnp.where(lane < TOPK, bdesc, NEG_INF)
            f0 = fbuf[r, pl.ds(0, 16)]
            f1 = fbuf[r, pl.ds(16, 16)]
            mx = jnp.maximum(jnp.maximum(f0, f1), top)
            for s in (1, 2, 4, 8):
                mx = jnp.maximum(mx, mx[jnp.bitwise_xor(lane, s)])
            e0 = jnp.exp(f0 - mx)
            e1 = jnp.exp(f1 - mx)
            et = jnp.exp(top - mx)
            es = e0 + e1 + et
            for s in (1, 2, 4, 8):
                es = es + es[jnp.bitwise_xor(lane, s)]
            inv = jnp.float32(1.0) / es
            o1buf[r, pl.ds(0, 16)] = e0 * inv
            o1buf[r, pl.ds(16, 16)] = e1 * inv
            o2buf[r, :] = et * inv
        pltpu.sync_copy(o1buf, out1_hbm.at[oct_i])
        pltpu.sync_copy(o2buf, out2_hbm.at[oct_i])


def kernel(item_scores, feat_scores, cand_item):
    del cand_item
    feat = jnp.pad(feat_scores.astype(jnp.float32),
                   ((0, 0), (0, 32 - N_FEAT)), constant_values=-jnp.inf)

    mesh = plsc.VectorSubcoreMesh(core_axis_name="c", subcore_axis_name="s")
    run = pl.kernel(
        _sc_body,
        mesh=mesh,
        out_type=[
            jax.ShapeDtypeStruct((N_OCT, 8, 32), jnp.float32),
            jax.ShapeDtypeStruct((N_OCT, 8, 16), jnp.float32),
        ],
        scratch_types=[
            pltpu.VMEM((8, CW), jnp.float32),
            pltpu.VMEM((8, CW), jnp.float32),
            pltpu.VMEM((8, REM), jnp.float32),
            pltpu.VMEM((CAND_CAP,), jnp.float32),
            pltpu.VMEM((8, 32), jnp.float32),
            pltpu.VMEM((8, 16), jnp.float32),
            pltpu.VMEM((8, 16), jnp.float32),
            pltpu.VMEM((8, 32), jnp.float32),
            pltpu.VMEM((8, 16), jnp.float32),
            pltpu.VMEM_SHARED((16, 8, 16), jnp.float32),
            pltpu.SemaphoreType.DMA,
            pltpu.SemaphoreType.DMA,
        ],
        compiler_params=pltpu.CompilerParams(needs_layout_passes=False),
    )
    out1, out2 = run(item_scores, feat)
    out1 = out1.reshape(B, 32)
    out2 = out2.reshape(B, 16)
    return jnp.concatenate([out1[:, :N_FEAT], out2[:, :TOPK]], axis=1)
